# baseline (speedup 1.0000x reference)
import jax
import jax.numpy as jnp
from jax.experimental import pallas as pl
from jax.experimental.pallas import tpu as pltpu


def _blk(x_ref, w_ref, o_ref):
    xb = x_ref[...].astype(jnp.bfloat16)
    o_ref[...] = jnp.dot(xb, w_ref[...], preferred_element_type=jnp.float32)


@jax.jit
def kernel(x, W):
    B, K = x.shape
    N = W.shape[1]
    W2 = jnp.concatenate([W, W], axis=1).astype(jnp.bfloat16)
    out = pl.pallas_call(
        _blk,
        grid=(8,),
        in_specs=[
            pl.BlockSpec((2048, K), lambda i: (i, 0)),
            pl.BlockSpec((K, 2 * N), lambda i: (0, 0)),
        ],
        out_specs=pl.BlockSpec((2048, 2 * N), lambda i: (i, 0)),
        out_shape=jax.ShapeDtypeStruct((B, 2 * N), jnp.float32),
        compiler_params=pltpu.CompilerParams(
            dimension_semantics=("arbitrary",),
        ),
    )(x, W2)
    return out[:, :N]


# speedup vs baseline: 1.1610x; 1.1610x over previous
import jax
import jax.numpy as jnp
from jax.experimental import pallas as pl
from jax.experimental.pallas import tpu as pltpu

TILE_B = 4096


def _blk(x_ref, w_ref, o_ref):
    o_ref[...] = jnp.dot(x_ref[...].astype(jnp.bfloat16), w_ref[...],
                         preferred_element_type=jnp.float32)


@jax.jit
def kernel(x, W):
    B, K = x.shape
    N = W.shape[1]
    return pl.pallas_call(
        _blk,
        grid=(B // TILE_B,),
        in_specs=[
            pl.BlockSpec((TILE_B, K), lambda i: (i, 0)),
            pl.BlockSpec((K, N), lambda i: (0, 0)),
        ],
        out_specs=pl.BlockSpec((TILE_B, N), lambda i: (i, 0)),
        out_shape=jax.ShapeDtypeStruct((B, N), jnp.float32),
        compiler_params=pltpu.CompilerParams(
            dimension_semantics=("arbitrary",),
        ),
    )(x, W.astype(jnp.bfloat16))


# speedup vs baseline: 1.2613x; 1.0863x over previous
import jax
import jax.numpy as jnp
from jax.experimental import pallas as pl
from jax.experimental.pallas import tpu as pltpu

TILE_B = 8192


def _blk(x_ref, w_ref, o_ref):
    o_ref[...] = jnp.dot(x_ref[...].astype(jnp.bfloat16), w_ref[...],
                         preferred_element_type=jnp.float32)


@jax.jit
def kernel(x, W):
    B, K = x.shape
    N = W.shape[1]
    return pl.pallas_call(
        _blk,
        grid=(B // TILE_B,),
        in_specs=[
            pl.BlockSpec((TILE_B, K), lambda i: (i, 0)),
            pl.BlockSpec((K, N), lambda i: (0, 0)),
        ],
        out_specs=pl.BlockSpec((TILE_B, N), lambda i: (i, 0)),
        out_shape=jax.ShapeDtypeStruct((B, N), jnp.float32),
        compiler_params=pltpu.CompilerParams(
            dimension_semantics=("arbitrary",),
        ),
    )(x, W.astype(jnp.bfloat16))


# speedup vs baseline: 1.2634x; 1.0017x over previous
import jax
import jax.numpy as jnp
from jax.experimental import pallas as pl
from jax.experimental.pallas import tpu as pltpu

TILE_B = 8192


def _step(x_ref, w_ref, o_ref, y_ref, sems):
    i = pl.program_id(0)
    y_ref[i] = jnp.dot(x_ref[...].astype(jnp.bfloat16), w_ref[...],
                       preferred_element_type=jnp.float32)
    pltpu.make_async_copy(
        y_ref.at[i], o_ref.at[pl.ds(i * TILE_B, TILE_B), :], sems.at[i]).start()

    @pl.when(i == 1)
    def _drain():
        for k in range(2):
            pltpu.make_async_copy(
                y_ref.at[k], o_ref.at[pl.ds(k * TILE_B, TILE_B), :],
                sems.at[k]).wait()


@jax.jit
def kernel(x, W):
    B, K = x.shape
    N = W.shape[1]
    return pl.pallas_call(
        _step,
        grid=(B // TILE_B,),
        in_specs=[
            pl.BlockSpec((TILE_B, K), lambda i: (i, 0)),
            pl.BlockSpec((K, N), lambda i: (0, 0)),
        ],
        out_specs=pl.BlockSpec(memory_space=pl.ANY),
        out_shape=jax.ShapeDtypeStruct((B, N), jnp.float32),
        scratch_shapes=[
            pltpu.VMEM((2, TILE_B, N), jnp.float32),
            pltpu.SemaphoreType.DMA((2,)),
        ],
        compiler_params=pltpu.CompilerParams(
            dimension_semantics=("arbitrary",),
        ),
    )(x, W.astype(jnp.bfloat16))
